# Initial kernel scaffold; baseline (speedup 1.0000x reference)
import jax
import jax.numpy as jnp
from jax.experimental import pallas as pl


def kernel(x, prototypes):
    raise NotImplementedError("write your pallas kernel here")


# speedup vs baseline: 124.4865x; 124.4865x over previous
import jax
import jax.numpy as jnp
from jax.experimental import pallas as pl


def _proto_topk_kernel(x_ref, pt_ref, dist_ref, idx_ref):
    xb = x_ref[0]
    pt = pt_ref[...]
    nrm = jnp.sqrt(jnp.sum(pt * pt, axis=0, keepdims=True))
    ptn = pt * (1.0 / jnp.maximum(nrm, 1e-12))
    s = jax.lax.dot_general(
        xb, ptn, (((1,), (0,)), ((), ())),
        preferred_element_type=jnp.float32,
    )
    m = jnp.max(s, axis=0, keepdims=True)
    iota = jax.lax.broadcasted_iota(jnp.int32, s.shape, 0)
    big = jnp.int32(s.shape[0])
    idx = jnp.min(jnp.where(s == m, iota, big), axis=0, keepdims=True)
    dist_ref[0] = 1.0 - m
    idx_ref[0] = idx


def kernel(x, prototypes):
    B, N, D = x.shape
    K = prototypes.shape[0]
    pt = prototypes.T
    dist, idx = pl.pallas_call(
        _proto_topk_kernel,
        grid=(B,),
        in_specs=[
            pl.BlockSpec((1, N, D), lambda b: (b, 0, 0)),
            pl.BlockSpec((D, K), lambda b: (0, 0)),
        ],
        out_specs=[
            pl.BlockSpec((1, 1, K), lambda b: (b, 0, 0)),
            pl.BlockSpec((1, 1, K), lambda b: (b, 0, 0)),
        ],
        out_shape=[
            jax.ShapeDtypeStruct((B, 1, K), jnp.float32),
            jax.ShapeDtypeStruct((B, 1, K), jnp.int32),
        ],
    )(x, pt)
    return dist[:, 0, :], idx[:, 0, :]


# speedup vs baseline: 133.6505x; 1.0736x over previous
import jax
import jax.numpy as jnp
from jax.experimental import pallas as pl
from jax.experimental.pallas import tpu as pltpu


def _proto_topk_kernel(x_ref, pt_ref, dist_ref, idx_ref, ptn_ref):
    b = pl.program_id(0)

    @pl.when(b == 0)
    def _normalize():
        pt = pt_ref[...]
        nrm = jnp.sqrt(jnp.sum(pt * pt, axis=0, keepdims=True))
        ptn_ref[...] = pt * (1.0 / jnp.maximum(nrm, 1e-12))

    xb = x_ref[0]
    s = jax.lax.dot_general(
        xb, ptn_ref[...], (((1,), (0,)), ((), ())),
        preferred_element_type=jnp.float32,
    )
    m = jnp.max(s, axis=0, keepdims=True)
    iota = jax.lax.broadcasted_iota(jnp.int32, s.shape, 0)
    big = jnp.int32(s.shape[0])
    idx = jnp.min(jnp.where(s == m, iota, big), axis=0, keepdims=True)
    dist_ref[0] = 1.0 - m
    idx_ref[0] = idx


def kernel(x, prototypes):
    B, N, D = x.shape
    K = prototypes.shape[0]
    pt = prototypes.T
    dist, idx = pl.pallas_call(
        _proto_topk_kernel,
        grid=(B,),
        in_specs=[
            pl.BlockSpec((1, N, D), lambda b: (b, 0, 0)),
            pl.BlockSpec((D, K), lambda b: (0, 0)),
        ],
        out_specs=[
            pl.BlockSpec((1, 1, K), lambda b: (b, 0, 0)),
            pl.BlockSpec((1, 1, K), lambda b: (b, 0, 0)),
        ],
        out_shape=[
            jax.ShapeDtypeStruct((B, 1, K), jnp.float32),
            jax.ShapeDtypeStruct((B, 1, K), jnp.int32),
        ],
        scratch_shapes=[pltpu.VMEM((D, K), jnp.float32)],
    )(x, pt)
    return dist[:, 0, :], idx[:, 0, :]


# speedup vs baseline: 154.8818x; 1.1589x over previous
import jax
import jax.numpy as jnp
from jax.experimental import pallas as pl
from jax.experimental.pallas import tpu as pltpu


def _proto_topk_kernel(x_ref, pt_ref, dist_ref, idx_ref, ptn_ref):
    b = pl.program_id(0)

    @pl.when(b == 0)
    def _normalize():
        pt = pt_ref[...]
        nrm = jnp.sqrt(jnp.sum(pt * pt, axis=0, keepdims=True))
        ptn_ref[...] = pt * (1.0 / jnp.maximum(nrm, 1e-12))

    xb = x_ref[0]
    s = jax.lax.dot_general(
        xb, ptn_ref[...], (((1,), (0,)), ((), ())),
        preferred_element_type=jnp.float32,
    )
    N, K = s.shape
    iota8 = jax.lax.broadcasted_iota(jnp.int32, (8, K), 0)
    run_m = s[0:8]
    run_i = iota8
    for i in range(1, N // 8):
        cur = s[8 * i:8 * i + 8]
        pred = cur > run_m
        run_m = jnp.where(pred, cur, run_m)
        run_i = jnp.where(pred, iota8 + jnp.int32(8 * i), run_i)
    m = jnp.max(run_m, axis=0, keepdims=True)
    big = jnp.int32(N)
    idx = jnp.min(jnp.where(run_m == m, run_i, big), axis=0, keepdims=True)
    dist_ref[0] = 1.0 - m
    idx_ref[0] = idx


def kernel(x, prototypes):
    B, N, D = x.shape
    K = prototypes.shape[0]
    pt = prototypes.T
    dist, idx = pl.pallas_call(
        _proto_topk_kernel,
        grid=(B,),
        in_specs=[
            pl.BlockSpec((1, N, D), lambda b: (b, 0, 0)),
            pl.BlockSpec((D, K), lambda b: (0, 0)),
        ],
        out_specs=[
            pl.BlockSpec((1, 1, K), lambda b: (b, 0, 0)),
            pl.BlockSpec((1, 1, K), lambda b: (b, 0, 0)),
        ],
        out_shape=[
            jax.ShapeDtypeStruct((B, 1, K), jnp.float32),
            jax.ShapeDtypeStruct((B, 1, K), jnp.int32),
        ],
        scratch_shapes=[pltpu.VMEM((D, K), jnp.float32)],
    )(x, pt)
    return dist[:, 0, :], idx[:, 0, :]
